# baseline (speedup 1.0000x reference)
import functools

import jax
import jax.numpy as jnp
from jax import lax
from jax.experimental import pallas as pl
from jax.experimental.pallas import tpu as pltpu
from jax.experimental.pallas import tpu_sc as plsc

V = 8192
HV = V // 2
B = 16
T = 1024
N = B * T

_NC = 2
_NS = 16
_L = 16
_NW = _NC * _NS
_PER_W = N // _NW
_CH = 4
_NCHUNK = _PER_W // _CH
_HALF = _NCHUNK // 2


def _mesh():
    return plsc.VectorSubcoreMesh(core_axis_name="c", subcore_axis_name="s")


def _sc_gather(table2, xh):
    @functools.partial(
        pl.kernel,
        mesh=_mesh(),
        out_type=jax.ShapeDtypeStruct((B, 2 * T, HV), jnp.float32),
        scratch_types=[
            pltpu.VMEM((2 * _PER_W,), jnp.int32),
            pltpu.VMEM((4 * _CH, HV), jnp.float32),
            pltpu.SemaphoreType.DMA,
            pltpu.SemaphoreType.DMA,
        ],
    )
    def k(table_hbm, xh_hbm, out_hbm, xhv, rows, sg0, sg1):
        wid = lax.axis_index("s") * _NC + lax.axis_index("c")
        bi = wid // 2
        hi0 = (wid % 2) * (2 * _PER_W)
        pltpu.sync_copy(xh_hbm.at[pl.ds(wid * 2 * _PER_W, 2 * _PER_W)], xhv)

        def buf(half):
            return rows.at[pl.ds(half * 2 * _CH, 2 * _CH)]

        def gather(j, half, sem):
            pltpu.async_copy(
                table_hbm.at[xhv.at[pl.ds(j * 2 * _CH, 2 * _CH)]], buf(half),
                sem)

        def gwait(j, half, sem):
            pltpu.make_async_copy(
                table_hbm.at[xhv.at[pl.ds(j * 2 * _CH, 2 * _CH)]], buf(half),
                sem).wait()

        def scatter(j, half):
            pltpu.sync_copy(
                buf(half), out_hbm.at[bi, pl.ds(hi0 + j * 2 * _CH, 2 * _CH)])

        gather(0, 0, sg0)

        def body(jj, carry):
            j0 = 2 * jj
            gwait(j0, 0, sg0)
            gather(j0 + 1, 1, sg1)
            scatter(j0, 0)
            gwait(j0 + 1, 1, sg1)

            @pl.when(jj < _HALF - 1)
            def _():
                gather(j0 + 2, 0, sg0)

            scatter(j0 + 1, 1)
            return carry

        lax.fori_loop(0, _HALF, body, 0)

    return k(table2, xh)


_LSE_ROWS = 256


def _lse_body(tbl_ref, out_ref):
    blk = tbl_ref[...]
    m = jnp.max(blk, axis=1, keepdims=True)
    s = jnp.sum(jnp.exp(blk - m), axis=1, keepdims=True)
    out_ref[...] = jnp.log(s) + m


def _table_lse(table):
    return pl.pallas_call(
        _lse_body,
        grid=(V // _LSE_ROWS,),
        in_specs=[pl.BlockSpec((_LSE_ROWS, V), lambda i: (i, 0))],
        out_specs=pl.BlockSpec((_LSE_ROWS, 1), lambda i: (i, 0)),
        out_shape=jax.ShapeDtypeStruct((V, 1), jnp.float32),
    )(table)


def _sc_combine(lse1, lgflat, xf, tf):
    @functools.partial(
        pl.kernel,
        mesh=_mesh(),
        out_type=jax.ShapeDtypeStruct((_NW, _L), jnp.float32),
        scratch_types=[
            pltpu.VMEM((_PER_W,), jnp.int32),
            pltpu.VMEM((_PER_W,), jnp.int32),
            pltpu.VMEM((_PER_W,), jnp.int32),
            pltpu.VMEM((_PER_W,), jnp.float32),
            pltpu.VMEM((_PER_W,), jnp.float32),
            pltpu.VMEM((_L,), jnp.float32),
            pltpu.SemaphoreType.DMA,
        ],
    )
    def k(lse_hbm, lg_hbm, x_hbm, t_hbm, out_hbm, xv, tv, fv, lv, gv, accv,
          sem):
        wid = lax.axis_index("s") * _NC + lax.axis_index("c")
        base = wid * _PER_W
        pltpu.sync_copy(x_hbm.at[pl.ds(base, _PER_W)], xv)
        pltpu.sync_copy(t_hbm.at[pl.ds(base, _PER_W)], tv)

        lanes = lax.iota(jnp.int32, _L)

        def mkflat(i, c):
            sl = pl.ds(i * _L, _L)
            fv[sl] = (base + i * _L + lanes) * V + tv[sl]
            return c

        lax.fori_loop(0, _PER_W // _L, mkflat, 0)

        def gchunk(j, c):
            sl = pl.ds(j * 128, 128)
            pltpu.async_copy(lse_hbm.at[xv.at[sl]], lv.at[sl], sem).wait()
            pltpu.async_copy(lg_hbm.at[fv.at[sl]], gv.at[sl], sem).wait()
            return c

        lax.fori_loop(0, _PER_W // 128, gchunk, 0)

        def red(i, acc):
            sl = pl.ds(i * _L, _L)
            return acc + (lv[sl] - gv[sl])

        acc = lax.fori_loop(0, _PER_W // _L, red, jnp.zeros((_L,), jnp.float32))
        accv[...] = acc
        pltpu.sync_copy(accv, out_hbm.at[wid])

    return k(lse1, lgflat, xf, tf)


def _loss_body(p_ref, o_ref):
    o_ref[...] = (jnp.sum(p_ref[...]) / N).reshape(1, 1)


def _loss_sum(partials):
    return pl.pallas_call(
        _loss_body,
        out_shape=jax.ShapeDtypeStruct((1, 1), jnp.float32),
    )(partials)


def kernel(x, targets, table):
    xf = x.reshape(-1)
    tf = targets.reshape(-1)
    xh = (2 * xf[:, None] + jnp.arange(2, dtype=jnp.int32)[None, :]).reshape(-1)
    table2 = table.reshape(2 * V, HV)
    logits2 = _sc_gather(table2, xh)
    lse1 = _table_lse(table).reshape(-1)
    partials = _sc_combine(lse1, logits2.reshape(-1), xf, tf)
    loss = _loss_sum(partials)[0, 0]
    return logits2.reshape(B, T, V), loss


# speedup vs baseline: 1.0337x; 1.0337x over previous
import functools

import jax
import jax.numpy as jnp
from jax import lax
from jax.experimental import pallas as pl
from jax.experimental.pallas import tpu as pltpu
from jax.experimental.pallas import tpu_sc as plsc

V = 8192
HV = V // 2
B = 16
T = 1024
N = B * T

_NC = 2
_NS = 16
_L = 16
_NW = _NC * _NS
_PER_W = N // _NW
_CH = 4
_NCHUNK = _PER_W // _CH
_HALF = _NCHUNK // 2


def _mesh():
    return plsc.VectorSubcoreMesh(core_axis_name="c", subcore_axis_name="s")


def _sc_gather(table2, xh):
    @functools.partial(
        pl.kernel,
        mesh=_mesh(),
        out_type=jax.ShapeDtypeStruct((B, 2 * T, HV), jnp.float32),
        scratch_types=[
            pltpu.VMEM((2 * _PER_W,), jnp.int32),
            pltpu.VMEM((4 * _CH, HV), jnp.float32),
            pltpu.SemaphoreType.DMA,
            pltpu.SemaphoreType.DMA,
        ],
    )
    def k(table_hbm, xh_hbm, out_hbm, xhv, rows, sg0, sg1):
        wid = lax.axis_index("s") * _NC + lax.axis_index("c")
        bi = wid // 2
        hi0 = (wid % 2) * (2 * _PER_W)
        pltpu.sync_copy(xh_hbm.at[pl.ds(wid * 2 * _PER_W, 2 * _PER_W)], xhv)

        def buf(half):
            return rows.at[pl.ds(half * 2 * _CH, 2 * _CH)]

        def gather(j, half, sem):
            pltpu.async_copy(
                table_hbm.at[xhv.at[pl.ds(j * 2 * _CH, 2 * _CH)]], buf(half),
                sem)

        def gwait(j, half, sem):
            pltpu.make_async_copy(
                table_hbm.at[xhv.at[pl.ds(j * 2 * _CH, 2 * _CH)]], buf(half),
                sem).wait()

        def scatter(j, half):
            pltpu.sync_copy(
                buf(half), out_hbm.at[bi, pl.ds(hi0 + j * 2 * _CH, 2 * _CH)])

        gather(0, 0, sg0)

        def body(jj, carry):
            j0 = 2 * jj
            gwait(j0, 0, sg0)
            gather(j0 + 1, 1, sg1)
            scatter(j0, 0)
            gwait(j0 + 1, 1, sg1)

            @pl.when(jj < _HALF - 1)
            def _():
                gather(j0 + 2, 0, sg0)

            scatter(j0 + 1, 1)
            return carry

        lax.fori_loop(0, _HALF, body, 0)

    return k(table2, xh)


_LSE_ROWS = 256


def _lse_body(tbl_ref, out_ref):
    blk = tbl_ref[...]
    m = jnp.max(blk, axis=1, keepdims=True)
    s = jnp.sum(jnp.exp(blk - m), axis=1, keepdims=True)
    out_ref[...] = jnp.log(s) + m


def _table_lse(table):
    return pl.pallas_call(
        _lse_body,
        grid=(V // _LSE_ROWS,),
        in_specs=[pl.BlockSpec((_LSE_ROWS, V), lambda i: (i, 0))],
        out_specs=pl.BlockSpec((_LSE_ROWS, 1), lambda i: (i, 0)),
        out_shape=jax.ShapeDtypeStruct((V, 1), jnp.float32),
    )(table)


def _sc_combine(lse1, table128, xf, tf):
    @functools.partial(
        pl.kernel,
        mesh=_mesh(),
        out_type=[
            jax.ShapeDtypeStruct((_NW, _L), jnp.float32),
            jax.ShapeDtypeStruct((N, 128), jnp.float32),
        ],
        scratch_types=[
            pltpu.VMEM((_PER_W,), jnp.int32),
            pltpu.VMEM((_PER_W,), jnp.int32),
            pltpu.VMEM((_PER_W,), jnp.int32),
            pltpu.VMEM((_PER_W,), jnp.float32),
            pltpu.VMEM((_PER_W, 128), jnp.float32),
            pltpu.VMEM((_L,), jnp.float32),
            pltpu.SemaphoreType.DMA,
        ],
    )
    def k(lse_hbm, t128_hbm, x_hbm, t_hbm, out_hbm, sl_hbm, xv, tv, fv, lv,
          gv, accv, sem):
        wid = lax.axis_index("s") * _NC + lax.axis_index("c")
        base = wid * _PER_W
        pltpu.sync_copy(x_hbm.at[pl.ds(base, _PER_W)], xv)
        pltpu.sync_copy(t_hbm.at[pl.ds(base, _PER_W)], tv)

        def mkflat(i, c):
            sl = pl.ds(i * _L, _L)
            fv[sl] = xv[sl] * (V // 128) + jnp.right_shift(tv[sl], 7)
            return c

        lax.fori_loop(0, _PER_W // _L, mkflat, 0)

        def gchunk(j, c):
            sl = pl.ds(j * 128, 128)
            pltpu.async_copy(lse_hbm.at[xv.at[sl]], lv.at[sl], sem).wait()
            pltpu.async_copy(t128_hbm.at[fv.at[sl]], gv.at[sl], sem).wait()
            return c

        lax.fori_loop(0, _PER_W // 128, gchunk, 0)

        def red(i, acc):
            return acc + lv[pl.ds(i * _L, _L)]

        acc = lax.fori_loop(0, _PER_W // _L, red, jnp.zeros((_L,), jnp.float32))
        accv[...] = acc
        pltpu.sync_copy(accv, out_hbm.at[wid])
        pltpu.sync_copy(gv, sl_hbm.at[pl.ds(base, _PER_W)])

    return k(lse1, table128, xf, tf)


def _loss_body(p_ref, s_ref, t_ref, o_ref):
    lanes = jax.lax.broadcasted_iota(jnp.int32, (N, 128), 1)
    sel = jnp.where(lanes == jnp.bitwise_and(t_ref[...], 127), s_ref[...], 0.0)
    o_ref[...] = ((jnp.sum(p_ref[...]) - jnp.sum(sel)) / N).reshape(1, 1)


def _loss_sum(lse_parts, slivers, t2):
    return pl.pallas_call(
        _loss_body,
        out_shape=jax.ShapeDtypeStruct((1, 1), jnp.float32),
    )(lse_parts, slivers, t2)


def kernel(x, targets, table):
    xf = x.reshape(-1)
    tf = targets.reshape(-1)
    xh = (2 * xf[:, None] + jnp.arange(2, dtype=jnp.int32)[None, :]).reshape(-1)
    table2 = table.reshape(2 * V, HV)
    table128 = table.reshape(V * V // 128, 128)
    logits2 = _sc_gather(table2, xh)
    lse1 = _table_lse(table).reshape(-1)
    lse_parts, slivers = _sc_combine(lse1, table128, xf, tf)
    loss = _loss_sum(lse_parts, slivers, tf.reshape(N, 1))[0, 0]
    return logits2.reshape(B, T, V), loss


# speedup vs baseline: 2.0132x; 1.9476x over previous
import functools

import jax
import jax.numpy as jnp
from jax import lax
from jax.experimental import pallas as pl
from jax.experimental.pallas import tpu as pltpu
from jax.experimental.pallas import tpu_sc as plsc

V = 8192
HV = V // 2
B = 16
T = 1024
N = B * T

_NC = 2
_NS = 16
_L = 16
_NW = _NC * _NS
_PER_W = N // _NW
_CH = 4
_NCHUNK = _PER_W // _CH
_HALF = _NCHUNK // 2


def _mesh():
    return plsc.VectorSubcoreMesh(core_axis_name="c", subcore_axis_name="s")


_GCH = 8
_NGCH = _PER_W // _GCH


def _sc_gather(table, xf):
    @functools.partial(
        pl.kernel,
        mesh=_mesh(),
        out_type=jax.ShapeDtypeStruct((N, V), jnp.float32),
        scratch_types=[
            pltpu.VMEM((_PER_W,), jnp.int32),
            pltpu.VMEM((_GCH, V), jnp.float32),
            pltpu.SemaphoreType.DMA,
        ],
    )
    def k(table_hbm, x_hbm, out_hbm, xv, rows, sem):
        wid = lax.axis_index("s") * _NC + lax.axis_index("c")
        base = wid * _PER_W
        pltpu.sync_copy(x_hbm.at[pl.ds(base, _PER_W)], xv)

        def chunk(j, carry):
            off = j * _GCH
            pltpu.async_copy(
                table_hbm.at[xv.at[pl.ds(off, _GCH)]], rows, sem).wait()
            pltpu.sync_copy(rows, out_hbm.at[pl.ds(base + off, _GCH)])
            return carry

        lax.fori_loop(0, _NGCH, chunk, 0)

    return k(table, xf)


_LSE_ROWS = 256


def _lse_body(tbl_ref, out_ref):
    blk = tbl_ref[...]
    m = jnp.max(blk, axis=1, keepdims=True)
    s = jnp.sum(jnp.exp(blk - m), axis=1, keepdims=True)
    out_ref[...] = jnp.log(s) + m


def _table_lse(table):
    return pl.pallas_call(
        _lse_body,
        grid=(V // _LSE_ROWS,),
        in_specs=[pl.BlockSpec((_LSE_ROWS, V), lambda i: (i, 0))],
        out_specs=pl.BlockSpec((_LSE_ROWS, 1), lambda i: (i, 0)),
        out_shape=jax.ShapeDtypeStruct((V, 1), jnp.float32),
    )(table)


def _sc_combine(lse1, table128, xf, tf):
    @functools.partial(
        pl.kernel,
        mesh=_mesh(),
        out_type=[
            jax.ShapeDtypeStruct((_NW, _L), jnp.float32),
            jax.ShapeDtypeStruct((N, 128), jnp.float32),
        ],
        scratch_types=[
            pltpu.VMEM((_PER_W,), jnp.int32),
            pltpu.VMEM((_PER_W,), jnp.int32),
            pltpu.VMEM((_PER_W,), jnp.int32),
            pltpu.VMEM((_PER_W,), jnp.float32),
            pltpu.VMEM((_PER_W, 128), jnp.float32),
            pltpu.VMEM((_L,), jnp.float32),
            pltpu.SemaphoreType.DMA,
        ],
    )
    def k(lse_hbm, t128_hbm, x_hbm, t_hbm, out_hbm, sl_hbm, xv, tv, fv, lv,
          gv, accv, sem):
        wid = lax.axis_index("s") * _NC + lax.axis_index("c")
        base = wid * _PER_W
        pltpu.sync_copy(x_hbm.at[pl.ds(base, _PER_W)], xv)
        pltpu.sync_copy(t_hbm.at[pl.ds(base, _PER_W)], tv)

        def mkflat(i, c):
            sl = pl.ds(i * _L, _L)
            fv[sl] = xv[sl] * (V // 128) + jnp.right_shift(tv[sl], 7)
            return c

        lax.fori_loop(0, _PER_W // _L, mkflat, 0)

        def gchunk(j, c):
            sl = pl.ds(j * 128, 128)
            pltpu.async_copy(lse_hbm.at[xv.at[sl]], lv.at[sl], sem).wait()
            pltpu.async_copy(t128_hbm.at[fv.at[sl]], gv.at[sl], sem).wait()
            return c

        lax.fori_loop(0, _PER_W // 128, gchunk, 0)

        def red(i, acc):
            return acc + lv[pl.ds(i * _L, _L)]

        acc = lax.fori_loop(0, _PER_W // _L, red, jnp.zeros((_L,), jnp.float32))
        accv[...] = acc
        pltpu.sync_copy(accv, out_hbm.at[wid])
        pltpu.sync_copy(gv, sl_hbm.at[pl.ds(base, _PER_W)])

    return k(lse1, table128, xf, tf)


def _loss_body(p_ref, s_ref, t_ref, o_ref):
    lanes = jax.lax.broadcasted_iota(jnp.int32, (N, 128), 1)
    sel = jnp.where(lanes == jnp.bitwise_and(t_ref[...], 127), s_ref[...], 0.0)
    o_ref[...] = ((jnp.sum(p_ref[...]) - jnp.sum(sel)) / N).reshape(1, 1)


def _loss_sum(lse_parts, slivers, t2):
    return pl.pallas_call(
        _loss_body,
        out_shape=jax.ShapeDtypeStruct((1, 1), jnp.float32),
    )(lse_parts, slivers, t2)


def kernel(x, targets, table):
    xf = x.reshape(-1)
    tf = targets.reshape(-1)
    table128 = table.reshape(V * V // 128, 128)
    logits = _sc_gather(table, xf)
    lse1 = _table_lse(table).reshape(-1)
    lse_parts, slivers = _sc_combine(lse1, table128, xf, tf)
    loss = _loss_sum(lse_parts, slivers, tf.reshape(N, 1))[0, 0]
    return logits.reshape(B, T, V), loss


# speedup vs baseline: 2.3540x; 1.1693x over previous
import functools

import jax
import jax.numpy as jnp
from jax import lax
from jax.experimental import pallas as pl
from jax.experimental.pallas import tpu as pltpu
from jax.experimental.pallas import tpu_sc as plsc

V = 8192
HV = V // 2
B = 16
T = 1024
N = B * T

_NC = 2
_NS = 16
_L = 16
_NW = _NC * _NS
_PER_W = N // _NW
_CH = 4
_NCHUNK = _PER_W // _CH
_HALF = _NCHUNK // 2


def _mesh():
    return plsc.VectorSubcoreMesh(core_axis_name="c", subcore_axis_name="s")


_GCH = 8
_NGCH = _PER_W // _GCH


def _sc_gather(table, xf):
    @functools.partial(
        pl.kernel,
        mesh=_mesh(),
        out_type=[
            jax.ShapeDtypeStruct((N, V), jnp.float32),
            jax.ShapeDtypeStruct((_NW, _L), jnp.float32),
        ],
        scratch_types=[
            pltpu.VMEM((_PER_W,), jnp.int32),
            pltpu.VMEM((_GCH, V), jnp.float32),
            pltpu.VMEM((_L,), jnp.float32),
            pltpu.SemaphoreType.DMA,
        ],
    )
    def k(table_hbm, x_hbm, out_hbm, tok_hbm, xv, rows, tokv, sem):
        wid = lax.axis_index("s") * _NC + lax.axis_index("c")
        base = wid * _PER_W
        pltpu.sync_copy(x_hbm.at[pl.ds(base, _PER_W)], xv)

        def chunk(j, carry):
            off = j * _GCH
            pltpu.async_copy(
                table_hbm.at[xv.at[pl.ds(off, _GCH)]], rows, sem).wait()
            pltpu.sync_copy(rows, out_hbm.at[pl.ds(base + off, _GCH)])
            return carry

        lax.fori_loop(0, _NGCH, chunk, 0)
        tokv[...] = jnp.zeros((_L,), jnp.float32)
        pltpu.sync_copy(tokv, tok_hbm.at[wid])

    return k(table, xf)


_LSE_ROWS = 256


def _lse_body(tbl_ref, out_ref):
    blk = tbl_ref[...]
    m = jnp.max(blk, axis=1, keepdims=True)
    s = jnp.sum(jnp.exp(blk - m), axis=1, keepdims=True)
    out_ref[...] = jnp.log(s) + m


def _table_lse(table):
    return pl.pallas_call(
        _lse_body,
        grid=(V // _LSE_ROWS,),
        in_specs=[pl.BlockSpec((_LSE_ROWS, V), lambda i: (i, 0))],
        out_specs=pl.BlockSpec((_LSE_ROWS, 1), lambda i: (i, 0)),
        out_shape=jax.ShapeDtypeStruct((V, 1), jnp.float32),
    )(table)


def _sc_combine(lse1, table128, xf, tf, tok):
    @functools.partial(
        pl.kernel,
        mesh=_mesh(),
        out_type=[
            jax.ShapeDtypeStruct((_NW, _L), jnp.float32),
            jax.ShapeDtypeStruct((N, 128), jnp.float32),
        ],
        scratch_types=[
            pltpu.VMEM((_PER_W,), jnp.int32),
            pltpu.VMEM((_PER_W,), jnp.int32),
            pltpu.VMEM((_PER_W,), jnp.int32),
            pltpu.VMEM((_PER_W,), jnp.float32),
            pltpu.VMEM((_PER_W, 128), jnp.float32),
            pltpu.VMEM((_L,), jnp.float32),
            pltpu.SemaphoreType.DMA,
        ],
    )
    def k(lse_hbm, t128_hbm, x_hbm, t_hbm, tok_hbm, out_hbm, sl_hbm, xv, tv,
          fv, lv, gv, accv, sem):
        wid = lax.axis_index("s") * _NC + lax.axis_index("c")
        base = wid * _PER_W
        pltpu.sync_copy(x_hbm.at[pl.ds(base, _PER_W)], xv)
        pltpu.sync_copy(t_hbm.at[pl.ds(base, _PER_W)], tv)

        def mkflat(i, c):
            sl = pl.ds(i * _L, _L)
            fv[sl] = xv[sl] * (V // 128) + jnp.right_shift(tv[sl], 7)
            return c

        lax.fori_loop(0, _PER_W // _L, mkflat, 0)

        def gchunk(j, c):
            sl = pl.ds(j * 128, 128)
            pltpu.async_copy(lse_hbm.at[xv.at[sl]], lv.at[sl], sem).wait()
            pltpu.async_copy(t128_hbm.at[fv.at[sl]], gv.at[sl], sem).wait()
            return c

        lax.fori_loop(0, _PER_W // 128, gchunk, 0)

        def red(i, acc):
            return acc + lv[pl.ds(i * _L, _L)]

        acc = lax.fori_loop(0, _PER_W // _L, red, jnp.zeros((_L,), jnp.float32))
        accv[...] = acc
        pltpu.sync_copy(accv, out_hbm.at[wid])
        pltpu.sync_copy(gv, sl_hbm.at[pl.ds(base, _PER_W)])

    return k(lse1, table128, xf, tf, tok)


def _loss_body(p_ref, s_ref, t_ref, o_ref):
    lanes = jax.lax.broadcasted_iota(jnp.int32, (N, 128), 1)
    sel = jnp.where(lanes == jnp.bitwise_and(t_ref[...], 127), s_ref[...], 0.0)
    o_ref[...] = ((jnp.sum(p_ref[...]) - jnp.sum(sel)) / N).reshape(1, 1)


def _loss_sum(lse_parts, slivers, t2):
    return pl.pallas_call(
        _loss_body,
        out_shape=jax.ShapeDtypeStruct((1, 1), jnp.float32),
    )(lse_parts, slivers, t2)


def kernel(x, targets, table):
    xf = x.reshape(-1)
    tf = targets.reshape(-1)
    table128 = table.reshape(V * V // 128, 128)
    logits, tok = _sc_gather(table, xf)
    lse1 = _table_lse(table).reshape(-1)
    lse_parts, slivers = _sc_combine(lse1, table128, xf, tf, tok)
    loss = _loss_sum(lse_parts, slivers, tf.reshape(N, 1))[0, 0]
    return logits.reshape(B, T, V), loss


# speedup vs baseline: 2.7782x; 1.1802x over previous
import functools

import jax
import jax.numpy as jnp
from jax import lax
from jax.experimental import pallas as pl
from jax.experimental.pallas import tpu as pltpu
from jax.experimental.pallas import tpu_sc as plsc

V = 8192
HV = V // 2
B = 16
T = 1024
N = B * T

_NC = 2
_NS = 16
_L = 16
_NW = _NC * _NS
_PER_W = N // _NW
_CH = 4
_NCHUNK = _PER_W // _CH
_HALF = _NCHUNK // 2


def _mesh():
    return plsc.VectorSubcoreMesh(core_axis_name="c", subcore_axis_name="s")


_GCH = 8
_NGCH = _PER_W // _GCH


def _sc_gather(table, xf):
    @functools.partial(
        pl.kernel,
        mesh=_mesh(),
        out_type=[
            jax.ShapeDtypeStruct((N, V), jnp.float32),
            jax.ShapeDtypeStruct((_NW, _L), jnp.float32),
        ],
        scratch_types=[
            pltpu.VMEM((_PER_W,), jnp.int32),
            pltpu.VMEM((_GCH, V), jnp.float32),
            pltpu.VMEM((_L,), jnp.float32),
            pltpu.SemaphoreType.DMA,
        ],
    )
    def k(table_hbm, x_hbm, out_hbm, tok_hbm, xv, rows, tokv, sem):
        wid = lax.axis_index("s") * _NC + lax.axis_index("c")
        base = wid * _PER_W
        pltpu.sync_copy(x_hbm.at[pl.ds(base, _PER_W)], xv)

        def chunk(j, carry):
            off = j * _GCH
            pltpu.async_copy(
                table_hbm.at[xv.at[pl.ds(off, _GCH)]], rows, sem).wait()
            pltpu.sync_copy(rows, out_hbm.at[pl.ds(base + off, _GCH)])
            return carry

        lax.fori_loop(0, _NGCH, chunk, 0)
        tokv[...] = jnp.zeros((_L,), jnp.float32)
        pltpu.sync_copy(tokv, tok_hbm.at[wid])

    return k(table, xf)


_LSE_ROWS = 256


def _lse_body(tbl_ref, out_ref, det_ref):
    blk = tbl_ref[...]
    m = jnp.max(blk, axis=1, keepdims=True)
    s = jnp.sum(jnp.exp(blk - m), axis=1, keepdims=True)
    out_ref[...] = jnp.log(s) + m
    det_ref[...] = blk.reshape(_LSE_ROWS, V // 128, 128)


def _table_lse(table):
    return pl.pallas_call(
        _lse_body,
        grid=(V // _LSE_ROWS,),
        in_specs=[pl.BlockSpec((_LSE_ROWS, V), lambda i: (i, 0))],
        out_specs=[
            pl.BlockSpec((_LSE_ROWS, 1), lambda i: (i, 0)),
            pl.BlockSpec((_LSE_ROWS, V // 128, 128), lambda i: (i, 0, 0)),
        ],
        out_shape=[
            jax.ShapeDtypeStruct((V, 1), jnp.float32),
            jax.ShapeDtypeStruct((V, V // 128, 128), jnp.float32),
        ],
    )(table)


def _sc_combine(lse1, table128, xf, tf, tok):
    @functools.partial(
        pl.kernel,
        mesh=_mesh(),
        out_type=[
            jax.ShapeDtypeStruct((_NW, _L), jnp.float32),
            jax.ShapeDtypeStruct((N, 128), jnp.float32),
        ],
        scratch_types=[
            pltpu.VMEM((_PER_W,), jnp.int32),
            pltpu.VMEM((_PER_W,), jnp.int32),
            pltpu.VMEM((_PER_W,), jnp.int32),
            pltpu.VMEM((_PER_W,), jnp.float32),
            pltpu.VMEM((_PER_W // 128, 128, 128), jnp.float32),
            pltpu.VMEM((_L,), jnp.float32),
            pltpu.SemaphoreType.DMA,
        ],
    )
    def k(lse_hbm, t128_hbm, x_hbm, t_hbm, tok_hbm, out_hbm, sl_hbm, xv, tv,
          fv, lv, gv, accv, sem):
        wid = lax.axis_index("s") * _NC + lax.axis_index("c")
        base = wid * _PER_W
        pltpu.sync_copy(x_hbm.at[pl.ds(base, _PER_W)], xv)
        pltpu.sync_copy(t_hbm.at[pl.ds(base, _PER_W)], tv)

        def mkflat(i, c):
            sl = pl.ds(i * _L, _L)
            fv[sl] = xv[sl] * (V // 128) + jnp.right_shift(tv[sl], 7)
            return c

        lax.fori_loop(0, _PER_W // _L, mkflat, 0)

        def gchunk(j, c):
            sl = pl.ds(j * 128, 128)
            pltpu.async_copy(lse_hbm.at[xv.at[sl]], lv.at[sl], sem).wait()
            pltpu.async_copy(t128_hbm.at[fv.at[sl]], gv.at[j], sem).wait()
            pltpu.sync_copy(gv.at[j], sl_hbm.at[pl.ds(base + j * 128, 128)])
            return c

        lax.fori_loop(0, _PER_W // 128, gchunk, 0)

        def red(i, acc):
            return acc + lv[pl.ds(i * _L, _L)]

        acc = lax.fori_loop(0, _PER_W // _L, red, jnp.zeros((_L,), jnp.float32))
        accv[...] = acc
        pltpu.sync_copy(accv, out_hbm.at[wid])

    return k(lse1, table128, xf, tf, tok)


def _loss_body(p_ref, s_ref, t_ref, o_ref):
    lanes = jax.lax.broadcasted_iota(jnp.int32, (N, 128), 1)
    sel = jnp.where(lanes == jnp.bitwise_and(t_ref[...], 127), s_ref[...], 0.0)
    o_ref[...] = ((jnp.sum(p_ref[...]) - jnp.sum(sel)) / N).reshape(1, 1)


def _loss_sum(lse_parts, slivers, t2):
    return pl.pallas_call(
        _loss_body,
        out_shape=jax.ShapeDtypeStruct((1, 1), jnp.float32),
    )(lse_parts, slivers, t2)


def kernel(x, targets, table):
    xf = x.reshape(-1)
    tf = targets.reshape(-1)
    logits, tok = _sc_gather(table, xf)
    lse, det = _table_lse(table)
    table128 = det.reshape(V * V // 128, 128)
    lse_parts, slivers = _sc_combine(lse.reshape(-1), table128, xf, tf, tok)
    loss = _loss_sum(lse_parts, slivers, tf.reshape(N, 1))[0, 0]
    return logits.reshape(B, T, V), loss


# speedup vs baseline: 2.9137x; 1.0488x over previous
import functools

import jax
import jax.numpy as jnp
from jax import lax
from jax.experimental import pallas as pl
from jax.experimental.pallas import tpu as pltpu
from jax.experimental.pallas import tpu_sc as plsc

V = 8192
HV = V // 2
B = 16
T = 1024
N = B * T

_NC = 2
_NS = 16
_L = 16
_NW = _NC * _NS
_PER_W = N // _NW
_CH = 4
_NCHUNK = _PER_W // _CH
_HALF = _NCHUNK // 2


def _mesh():
    return plsc.VectorSubcoreMesh(core_axis_name="c", subcore_axis_name="s")


_GCH = 8
_NGCH = _PER_W // _GCH


def _sc_gather(table, xf):
    @functools.partial(
        pl.kernel,
        mesh=_mesh(),
        out_type=[
            jax.ShapeDtypeStruct((N, V), jnp.float32),
            jax.ShapeDtypeStruct((_NW, _L), jnp.float32),
        ],
        scratch_types=[
            pltpu.VMEM((_PER_W,), jnp.int32),
            pltpu.VMEM((_GCH, V), jnp.float32),
            pltpu.VMEM((_L,), jnp.float32),
            pltpu.SemaphoreType.DMA,
        ],
    )
    def k(table_hbm, x_hbm, out_hbm, tok_hbm, xv, rows, tokv, sem):
        wid = lax.axis_index("s") * _NC + lax.axis_index("c")
        base = wid * _PER_W
        pltpu.sync_copy(x_hbm.at[pl.ds(base, _PER_W)], xv)

        def chunk(j, carry):
            off = j * _GCH
            pltpu.async_copy(
                table_hbm.at[xv.at[pl.ds(off, _GCH)]], rows, sem).wait()
            pltpu.sync_copy(rows, out_hbm.at[pl.ds(base + off, _GCH)])
            return carry

        lax.fori_loop(0, _NGCH, chunk, 0)
        tokv[...] = jnp.zeros((_L,), jnp.float32)
        pltpu.sync_copy(tokv, tok_hbm.at[wid])

    return k(table, xf)


_LSE_ROWS = 256


def _lse_body(tbl_ref, out_ref, det_ref):
    blk = tbl_ref[...]
    m = jnp.max(blk, axis=1, keepdims=True)
    s = jnp.sum(jnp.exp(blk - m), axis=1, keepdims=True)
    out_ref[...] = jnp.log(s) + m
    fb = jax.lax.bitcast_convert_type(blk, jnp.int32)
    rb = jnp.right_shift(
        fb + 0x7FFF + jnp.bitwise_and(jnp.right_shift(fb, 16), 1), 16)
    rb3 = rb.reshape(_LSE_ROWS // 2, 2, V)
    lo = rb3[:, 0, :]
    hi = rb3[:, 1, :]
    pk = jnp.bitwise_or(jnp.bitwise_and(lo, 0xFFFF),
                        jnp.left_shift(hi, 16))
    det_ref[...] = pk.reshape(_LSE_ROWS // 2, V // 128, 128)


def _table_lse(table):
    return pl.pallas_call(
        _lse_body,
        grid=(V // _LSE_ROWS,),
        in_specs=[pl.BlockSpec((_LSE_ROWS, V), lambda i: (i, 0))],
        out_specs=[
            pl.BlockSpec((_LSE_ROWS, 1), lambda i: (i, 0)),
            pl.BlockSpec((_LSE_ROWS // 2, V // 128, 128), lambda i: (i, 0, 0)),
        ],
        out_shape=[
            jax.ShapeDtypeStruct((V, 1), jnp.float32),
            jax.ShapeDtypeStruct((V // 2, V // 128, 128), jnp.int32),
        ],
    )(table)


def _sc_combine(lse1, table128, xf, tf, tok):
    @functools.partial(
        pl.kernel,
        mesh=_mesh(),
        out_type=[
            jax.ShapeDtypeStruct((_NW, _L), jnp.float32),
            jax.ShapeDtypeStruct((N, 128), jnp.int32),
        ],
        scratch_types=[
            pltpu.VMEM((_PER_W,), jnp.int32),
            pltpu.VMEM((_PER_W,), jnp.int32),
            pltpu.VMEM((_PER_W,), jnp.int32),
            pltpu.VMEM((_PER_W,), jnp.float32),
            pltpu.VMEM((_PER_W // 128, 128, 128), jnp.int32),
            pltpu.VMEM((_L,), jnp.float32),
            pltpu.SemaphoreType.DMA,
        ],
    )
    def k(lse_hbm, t128_hbm, x_hbm, t_hbm, tok_hbm, out_hbm, sl_hbm, xv, tv,
          fv, lv, gv, accv, sem):
        wid = lax.axis_index("s") * _NC + lax.axis_index("c")
        base = wid * _PER_W
        pltpu.sync_copy(x_hbm.at[pl.ds(base, _PER_W)], xv)
        pltpu.sync_copy(t_hbm.at[pl.ds(base, _PER_W)], tv)

        def mkflat(i, c):
            sl = pl.ds(i * _L, _L)
            fv[sl] = jnp.right_shift(xv[sl], 1) * (V // 128) \
                + jnp.right_shift(tv[sl], 7)
            return c

        lax.fori_loop(0, _PER_W // _L, mkflat, 0)

        def gchunk(j, c):
            sl = pl.ds(j * 128, 128)
            pltpu.async_copy(lse_hbm.at[xv.at[sl]], lv.at[sl], sem).wait()
            pltpu.async_copy(t128_hbm.at[fv.at[sl]], gv.at[j], sem).wait()
            pltpu.sync_copy(gv.at[j], sl_hbm.at[pl.ds(base + j * 128, 128)])
            return c

        lax.fori_loop(0, _PER_W // 128, gchunk, 0)

        def red(i, acc):
            return acc + lv[pl.ds(i * _L, _L)]

        acc = lax.fori_loop(0, _PER_W // _L, red, jnp.zeros((_L,), jnp.float32))
        accv[...] = acc
        pltpu.sync_copy(accv, out_hbm.at[wid])

    return k(lse1, table128, xf, tf, tok)


def _loss_body(p_ref, s_ref, t_ref, x_ref, o_ref):
    lanes = jax.lax.broadcasted_iota(jnp.int32, (N, 128), 1)
    t = t_ref[...]
    s = s_ref[...]
    bits = jnp.where(jnp.bitwise_and(x_ref[...], 1) == 1,
                     jnp.right_shift(s, 16), s)
    vals = jax.lax.bitcast_convert_type(
        jnp.left_shift(bits, 16), jnp.float32)
    sel = jnp.where(lanes == jnp.bitwise_and(t, 127), vals, 0.0)
    o_ref[...] = ((jnp.sum(p_ref[...]) - jnp.sum(sel)) / N).reshape(1, 1)


def _loss_sum(lse_parts, slivers, t2, x2):
    return pl.pallas_call(
        _loss_body,
        out_shape=jax.ShapeDtypeStruct((1, 1), jnp.float32),
    )(lse_parts, slivers, t2, x2)


def kernel(x, targets, table):
    xf = x.reshape(-1)
    tf = targets.reshape(-1)
    logits, tok = _sc_gather(table, xf)
    lse, det = _table_lse(table)
    table128 = det.reshape(V * V // 256, 128)
    lse_parts, slivers = _sc_combine(lse.reshape(-1), table128, xf, tf, tok)
    loss = _loss_sum(lse_parts, slivers, tf.reshape(N, 1),
                     xf.reshape(N, 1))[0, 0]
    return logits.reshape(B, T, V), loss


# speedup vs baseline: 3.0632x; 1.0513x over previous
import functools

import jax
import jax.numpy as jnp
from jax import lax
from jax.experimental import pallas as pl
from jax.experimental.pallas import tpu as pltpu
from jax.experimental.pallas import tpu_sc as plsc

V = 8192
HV = V // 2
B = 16
T = 1024
N = B * T

_NC = 2
_NS = 16
_L = 16
_NW = _NC * _NS
_PER_W = N // _NW
_CH = 4
_NCHUNK = _PER_W // _CH
_HALF = _NCHUNK // 2


def _mesh():
    return plsc.VectorSubcoreMesh(core_axis_name="c", subcore_axis_name="s")


_GCH = 8
_NGCH = _PER_W // _GCH


def _sc_gather(table, xf):
    @functools.partial(
        pl.kernel,
        mesh=_mesh(),
        out_type=[
            jax.ShapeDtypeStruct((N, V), jnp.float32),
            jax.ShapeDtypeStruct((_NW, _L), jnp.float32),
        ],
        scratch_types=[
            pltpu.VMEM((_PER_W,), jnp.int32),
            pltpu.VMEM((2 * _GCH, HV), jnp.float32),
            pltpu.VMEM((_L,), jnp.float32),
            pltpu.SemaphoreType.DMA,
            pltpu.SemaphoreType.DMA,
        ],
    )
    def k(table_hbm, x_hbm, out_hbm, tok_hbm, xv, rows, tokv, semL, semR):
        wid = lax.axis_index("s") * _NC + lax.axis_index("c")
        base = wid * _PER_W
        pltpu.sync_copy(x_hbm.at[pl.ds(base, _PER_W)], xv)

        bufL = rows.at[pl.ds(0, _GCH)]
        bufR = rows.at[pl.ds(_GCH, _GCH)]

        def gat(j, buf, c0, sem):
            pltpu.async_copy(
                table_hbm.at[xv.at[pl.ds(j * _GCH, _GCH)], pl.ds(c0, HV)],
                buf, sem)

        def gwait(j, buf, c0, sem):
            pltpu.make_async_copy(
                table_hbm.at[xv.at[pl.ds(j * _GCH, _GCH)], pl.ds(c0, HV)],
                buf, sem).wait()

        def sca(j, buf, c0):
            pltpu.sync_copy(
                buf, out_hbm.at[pl.ds(base + j * _GCH, _GCH), pl.ds(c0, HV)])

        gat(0, bufL, 0, semL)
        gat(0, bufR, HV, semR)

        def chunk(j, carry):
            gwait(j, bufL, 0, semL)
            sca(j, bufL, 0)
            gwait(j, bufR, HV, semR)

            @pl.when(j < _NGCH - 1)
            def _():
                gat(j + 1, bufL, 0, semL)

            sca(j, bufR, HV)

            @pl.when(j < _NGCH - 1)
            def _():
                gat(j + 1, bufR, HV, semR)

            return carry

        lax.fori_loop(0, _NGCH, chunk, 0)
        tokv[...] = jnp.zeros((_L,), jnp.float32)
        pltpu.sync_copy(tokv, tok_hbm.at[wid])

    return k(table, xf)


_LSE_ROWS = 256


def _lse_body(tbl_ref, out_ref, det_ref):
    blk = tbl_ref[...]
    m = jnp.max(blk, axis=1, keepdims=True)
    s = jnp.sum(jnp.exp(blk - m), axis=1, keepdims=True)
    out_ref[...] = jnp.log(s) + m
    fb = jax.lax.bitcast_convert_type(blk, jnp.int32)
    rb = jnp.right_shift(
        fb + 0x7FFF + jnp.bitwise_and(jnp.right_shift(fb, 16), 1), 16)
    rb3 = rb.reshape(_LSE_ROWS // 2, 2, V)
    lo = rb3[:, 0, :]
    hi = rb3[:, 1, :]
    pk = jnp.bitwise_or(jnp.bitwise_and(lo, 0xFFFF),
                        jnp.left_shift(hi, 16))
    det_ref[...] = pk.reshape(_LSE_ROWS // 2, V // 128, 128)


def _table_lse(table):
    return pl.pallas_call(
        _lse_body,
        grid=(V // _LSE_ROWS,),
        in_specs=[pl.BlockSpec((_LSE_ROWS, V), lambda i: (i, 0))],
        out_specs=[
            pl.BlockSpec((_LSE_ROWS, 1), lambda i: (i, 0)),
            pl.BlockSpec((_LSE_ROWS // 2, V // 128, 128), lambda i: (i, 0, 0)),
        ],
        out_shape=[
            jax.ShapeDtypeStruct((V, 1), jnp.float32),
            jax.ShapeDtypeStruct((V // 2, V // 128, 128), jnp.int32),
        ],
    )(table)


def _sc_combine(lse1, table128, xf, tf, tok):
    @functools.partial(
        pl.kernel,
        mesh=_mesh(),
        out_type=[
            jax.ShapeDtypeStruct((_NW, _L), jnp.float32),
            jax.ShapeDtypeStruct((N, 128), jnp.int32),
        ],
        scratch_types=[
            pltpu.VMEM((_PER_W,), jnp.int32),
            pltpu.VMEM((_PER_W,), jnp.int32),
            pltpu.VMEM((_PER_W,), jnp.int32),
            pltpu.VMEM((_PER_W,), jnp.float32),
            pltpu.VMEM((_PER_W // 128, 128, 128), jnp.int32),
            pltpu.VMEM((_L,), jnp.float32),
            pltpu.SemaphoreType.DMA,
        ],
    )
    def k(lse_hbm, t128_hbm, x_hbm, t_hbm, tok_hbm, out_hbm, sl_hbm, xv, tv,
          fv, lv, gv, accv, sem):
        wid = lax.axis_index("s") * _NC + lax.axis_index("c")
        base = wid * _PER_W
        pltpu.sync_copy(x_hbm.at[pl.ds(base, _PER_W)], xv)
        pltpu.sync_copy(t_hbm.at[pl.ds(base, _PER_W)], tv)

        def mkflat(i, c):
            sl = pl.ds(i * _L, _L)
            fv[sl] = jnp.right_shift(xv[sl], 1) * (V // 128) \
                + jnp.right_shift(tv[sl], 7)
            return c

        lax.fori_loop(0, _PER_W // _L, mkflat, 0)

        def gchunk(j, c):
            sl = pl.ds(j * 128, 128)
            pltpu.async_copy(lse_hbm.at[xv.at[sl]], lv.at[sl], sem).wait()
            pltpu.async_copy(t128_hbm.at[fv.at[sl]], gv.at[j], sem).wait()
            pltpu.sync_copy(gv.at[j], sl_hbm.at[pl.ds(base + j * 128, 128)])
            return c

        lax.fori_loop(0, _PER_W // 128, gchunk, 0)

        def red(i, acc):
            return acc + lv[pl.ds(i * _L, _L)]

        acc = lax.fori_loop(0, _PER_W // _L, red, jnp.zeros((_L,), jnp.float32))
        accv[...] = acc
        pltpu.sync_copy(accv, out_hbm.at[wid])

    return k(lse1, table128, xf, tf, tok)


def _loss_body(p_ref, s_ref, t_ref, x_ref, o_ref):
    lanes = jax.lax.broadcasted_iota(jnp.int32, (N, 128), 1)
    t = t_ref[...]
    s = s_ref[...]
    bits = jnp.where(jnp.bitwise_and(x_ref[...], 1) == 1,
                     jnp.right_shift(s, 16), s)
    vals = jax.lax.bitcast_convert_type(
        jnp.left_shift(bits, 16), jnp.float32)
    sel = jnp.where(lanes == jnp.bitwise_and(t, 127), vals, 0.0)
    o_ref[...] = ((jnp.sum(p_ref[...]) - jnp.sum(sel)) / N).reshape(1, 1)


def _loss_sum(lse_parts, slivers, t2, x2):
    return pl.pallas_call(
        _loss_body,
        out_shape=jax.ShapeDtypeStruct((1, 1), jnp.float32),
    )(lse_parts, slivers, t2, x2)


def kernel(x, targets, table):
    xf = x.reshape(-1)
    tf = targets.reshape(-1)
    logits, tok = _sc_gather(table, xf)
    lse, det = _table_lse(table)
    table128 = det.reshape(V * V // 256, 128)
    lse_parts, slivers = _sc_combine(lse.reshape(-1), table128, xf, tf, tok)
    loss = _loss_sum(lse_parts, slivers, tf.reshape(N, 1),
                     xf.reshape(N, 1))[0, 0]
    return logits.reshape(B, T, V), loss
